# Initial kernel scaffold; baseline (speedup 1.0000x reference)
import jax
import jax.numpy as jnp
from jax.experimental import pallas as pl


def kernel(boxes, classification, rotation, translation):
    raise NotImplementedError("write your pallas kernel here")


# speedup vs baseline: 10.1239x; 10.1239x over previous
import jax
import jax.numpy as jnp
from jax.experimental import pallas as pl
from jax.experimental.pallas import tpu as pltpu

_N = 20000
_C = 80
_K = 100
_THR = 0.01
_L = 128
_R = (_N * _C + _L - 1) // _L
_R = ((_R + 7) // 8) * 8
_PAD = _R * _L - _N * _C
_NEG = float("-inf")
_BIG = 2 ** 30


def _filter_kernel(scores_ref, feat_ref, featout_ref, scoreout_ref,
                   labelout_ref, work_ref, fid_ref):
    s = scores_ref[...]
    mask = s > _THR
    count = jnp.sum(mask.astype(jnp.int32))
    work_ref[...] = jnp.where(mask, s, _NEG)
    r_iota = jax.lax.broadcasted_iota(jnp.int32, (_R, _L), 0)
    l_iota = jax.lax.broadcasted_iota(jnp.int32, (_R, _L), 1)
    fid_ref[...] = r_iota * _L + l_iota

    lane_row = jax.lax.broadcasted_iota(jnp.int32, (1, 128), 1)
    a_col = jax.lax.broadcasted_iota(jnp.int32, (128, 1), 0)

    def body(j, carry):
        id_row, id_col, sc_row, sc_col = carry
        w = work_ref[...]
        m = jnp.max(w)
        fi = fid_ref[...]
        idx = jnp.min(jnp.where(w == m, fi, _BIG))
        work_ref[...] = jnp.where(fi == idx, _NEG, w)
        id_row = jnp.where(lane_row == j, idx, id_row)
        id_col = jnp.where(a_col == j, idx, id_col)
        sc_row = jnp.where(lane_row == j, m, sc_row)
        sc_col = jnp.where(a_col == j, m, sc_col)
        return id_row, id_col, sc_row, sc_col

    init = (jnp.zeros((1, 128), jnp.int32), jnp.zeros((128, 1), jnp.int32),
            jnp.full((1, 128), _NEG, jnp.float32),
            jnp.full((128, 1), _NEG, jnp.float32))
    id_row, id_col, sc_row, sc_col = jax.lax.fori_loop(0, _K, body, init)

    valid_row = lane_row < count
    valid_col = a_col < count
    rank_row = jnp.sum((valid_col & (id_col < id_row)).astype(jnp.int32),
                       axis=0, keepdims=True)
    place = (rank_row == a_col) & valid_row
    ordered_id_col = jnp.sum(jnp.where(place, id_row, 0), axis=1,
                             keepdims=True)
    ordered_sc_col = jnp.sum(jnp.where(place, sc_row, 0.0), axis=1,
                             keepdims=True)

    use_topk = count > _K
    fid_col = jnp.where(use_topk, id_col, ordered_id_col)
    fsc_col = jnp.where(use_topk, sc_col, ordered_sc_col)
    valid_out = a_col < jnp.minimum(count, _K)

    box_idx = fid_col % _N
    label = fid_col // _N
    i_row = jax.lax.broadcasted_iota(jnp.int32, (128, _N), 1)
    onehot = (box_idx == i_row).astype(jnp.float32)
    gat = jax.lax.dot_general(
        onehot, feat_ref[...], (((1,), (0,)), ((), ())),
        preferred_element_type=jnp.float32,
        precision=jax.lax.Precision.HIGHEST)
    featout_ref[...] = jnp.where(valid_out, gat, jnp.float32(-1.0))
    scoreout_ref[...] = jnp.where(valid_out, fsc_col, jnp.float32(-1.0))
    labelout_ref[...] = jnp.where(valid_out, label, jnp.int32(-1))


@jax.jit
def kernel(boxes, classification, rotation, translation):
    flat = classification.T.reshape(-1)
    flat = jnp.pad(flat, (0, _PAD), constant_values=-1.0)
    sc2d = flat.reshape(_R, _L)
    feat = jnp.concatenate([boxes, rotation, translation], axis=1)

    featout, scores, labels = pl.pallas_call(
        _filter_kernel,
        out_shape=[jax.ShapeDtypeStruct((128, 10), jnp.float32),
                   jax.ShapeDtypeStruct((128, 1), jnp.float32),
                   jax.ShapeDtypeStruct((128, 1), jnp.int32)],
        scratch_shapes=[pltpu.VMEM((_R, _L), jnp.float32),
                        pltpu.VMEM((_R, _L), jnp.int32)],
        compiler_params=pltpu.CompilerParams(
            vmem_limit_bytes=100 * 1024 * 1024),
    )(sc2d, feat)

    boxes_out = featout[:_K, :4]
    rotation_out = featout[:_K, 4:7]
    translation_out = featout[:_K, 7:10]
    scores_out = scores[:_K, 0]
    labels_out = labels[:_K, 0]
    return boxes_out, scores_out, labels_out, rotation_out, translation_out


# speedup vs baseline: 19.0704x; 1.8837x over previous
import jax
import jax.numpy as jnp
from jax.experimental import pallas as pl
from jax.experimental.pallas import tpu as pltpu

_N = 20000
_C = 80
_K = 100
_THR = 0.01
_L = 128
_S = 128
_B = 104
_R = _S * _B
_PAD = _R * _L - _N * _C
_NEG = float("-inf")
_BIG = 2 ** 30


def _filter_kernel(scores_ref, feat_ref, featout_ref, scoreout_ref,
                   labelout_ref, work_ref, segmax_ref):
    s = scores_ref[...]
    mask = s > _THR
    count = jnp.sum(mask.astype(jnp.int32))
    work_ref[...] = jnp.where(mask, s, _NEG)

    for seg in range(_S):
        segmax_ref[seg] = jnp.max(work_ref[seg * _B:(seg + 1) * _B, :])

    lane_row = jax.lax.broadcasted_iota(jnp.int32, (1, 128), 1)
    a_col = jax.lax.broadcasted_iota(jnp.int32, (128, 1), 0)
    r_seg = jax.lax.broadcasted_iota(jnp.int32, (_B, _L), 0)
    l_seg = jax.lax.broadcasted_iota(jnp.int32, (_B, _L), 1)

    def body(j, carry):
        id_row, id_col, sc_row, sc_col = carry

        def seg_scan(t, c):
            best_m, best_s = c
            v = segmax_ref[t]
            better = v > best_m
            return (jnp.where(better, v, best_m),
                    jnp.where(better, t, best_s))

        m, seg = jax.lax.fori_loop(
            0, _S, seg_scan, (jnp.float32(_NEG), jnp.int32(0)))

        base = seg * _B
        w = work_ref[pl.ds(base, _B), :]
        fi = (base + r_seg) * _L + l_seg
        idx = jnp.min(jnp.where(w == m, fi, _BIG))
        w = jnp.where(fi == idx, _NEG, w)
        work_ref[pl.ds(base, _B), :] = w
        segmax_ref[seg] = jnp.max(w)

        id_row = jnp.where(lane_row == j, idx, id_row)
        id_col = jnp.where(a_col == j, idx, id_col)
        sc_row = jnp.where(lane_row == j, m, sc_row)
        sc_col = jnp.where(a_col == j, m, sc_col)
        return id_row, id_col, sc_row, sc_col

    init = (jnp.zeros((1, 128), jnp.int32), jnp.zeros((128, 1), jnp.int32),
            jnp.full((1, 128), _NEG, jnp.float32),
            jnp.full((128, 1), _NEG, jnp.float32))
    id_row, id_col, sc_row, sc_col = jax.lax.fori_loop(0, _K, body, init)

    valid_row = lane_row < count
    valid_col = a_col < count
    rank_row = jnp.sum((valid_col & (id_col < id_row)).astype(jnp.int32),
                       axis=0, keepdims=True)
    place = (rank_row == a_col) & valid_row
    ordered_id_col = jnp.sum(jnp.where(place, id_row, 0), axis=1,
                             keepdims=True)
    ordered_sc_col = jnp.sum(jnp.where(place, sc_row, 0.0), axis=1,
                             keepdims=True)

    use_topk = count > _K
    fid_col = jnp.where(use_topk, id_col, ordered_id_col)
    fsc_col = jnp.where(use_topk, sc_col, ordered_sc_col)
    valid_out = a_col < jnp.minimum(count, _K)

    box_idx = fid_col % _N
    label = fid_col // _N
    i_row = jax.lax.broadcasted_iota(jnp.int32, (128, _N), 1)
    onehot = (box_idx == i_row).astype(jnp.float32)
    gat = jax.lax.dot_general(
        onehot, feat_ref[...], (((1,), (0,)), ((), ())),
        preferred_element_type=jnp.float32,
        precision=jax.lax.Precision.HIGHEST)
    featout_ref[...] = jnp.where(valid_out, gat, jnp.float32(-1.0))
    scoreout_ref[...] = jnp.where(valid_out, fsc_col, jnp.float32(-1.0))
    labelout_ref[...] = jnp.where(valid_out, label, jnp.int32(-1))


@jax.jit
def kernel(boxes, classification, rotation, translation):
    flat = classification.T.reshape(-1)
    flat = jnp.pad(flat, (0, _PAD), constant_values=-1.0)
    sc2d = flat.reshape(_R, _L)
    feat = jnp.concatenate([boxes, rotation, translation], axis=1)

    featout, scores, labels = pl.pallas_call(
        _filter_kernel,
        out_shape=[jax.ShapeDtypeStruct((128, 10), jnp.float32),
                   jax.ShapeDtypeStruct((128, 1), jnp.float32),
                   jax.ShapeDtypeStruct((128, 1), jnp.int32)],
        scratch_shapes=[pltpu.VMEM((_R, _L), jnp.float32),
                        pltpu.SMEM((_S,), jnp.float32)],
        compiler_params=pltpu.CompilerParams(
            vmem_limit_bytes=100 * 1024 * 1024),
    )(sc2d, feat)

    boxes_out = featout[:_K, :4]
    rotation_out = featout[:_K, 4:7]
    translation_out = featout[:_K, 7:10]
    scores_out = scores[:_K, 0]
    labels_out = labels[:_K, 0]
    return boxes_out, scores_out, labels_out, rotation_out, translation_out


# speedup vs baseline: 38.6664x; 2.0276x over previous
import jax
import jax.numpy as jnp
from jax.experimental import pallas as pl
from jax.experimental.pallas import tpu as pltpu

_N = 20000
_C = 80
_K = 100
_THR = 0.01
_L = 128
_S = 128
_B = 104
_R = _S * _B
_PAD = _R * _L - _N * _C
_NEG = float("-inf")
_BIG = 2 ** 30


def _filter_kernel(scores_ref, feat_ref, featout_ref, scoreout_ref,
                   labelout_ref, work_ref):
    lane_row = jax.lax.broadcasted_iota(jnp.int32, (1, 128), 1)
    a_col = jax.lax.broadcasted_iota(jnp.int32, (128, 1), 0)
    r_seg = jax.lax.broadcasted_iota(jnp.int32, (_B, _L), 0)
    l_seg = jax.lax.broadcasted_iota(jnp.int32, (_B, _L), 1)

    count = jnp.int32(0)
    segv = jnp.full((1, 128), _NEG, jnp.float32)
    for seg in range(_S):
        s = scores_ref[seg * _B:(seg + 1) * _B, :]
        mask = s > _THR
        count = count + jnp.sum(mask.astype(jnp.int32))
        w = jnp.where(mask, s, _NEG)
        work_ref[seg * _B:(seg + 1) * _B, :] = w
        segv = jnp.where(lane_row == seg, jnp.max(w), segv)

    def body(j, carry):
        id_row, id_col, sc_row, sc_col, segv = carry

        m = jnp.max(segv)
        seg = jnp.min(jnp.where(segv == m, lane_row, _BIG))

        base = seg * _B
        w = work_ref[pl.ds(base, _B), :]
        fi = (base + r_seg) * _L + l_seg
        idx = jnp.min(jnp.where(w == m, fi, _BIG))
        w = jnp.where(fi == idx, _NEG, w)
        work_ref[pl.ds(base, _B), :] = w
        segv = jnp.where(lane_row == seg, jnp.max(w), segv)

        id_row = jnp.where(lane_row == j, idx, id_row)
        id_col = jnp.where(a_col == j, idx, id_col)
        sc_row = jnp.where(lane_row == j, m, sc_row)
        sc_col = jnp.where(a_col == j, m, sc_col)
        return id_row, id_col, sc_row, sc_col, segv

    init = (jnp.zeros((1, 128), jnp.int32), jnp.zeros((128, 1), jnp.int32),
            jnp.full((1, 128), _NEG, jnp.float32),
            jnp.full((128, 1), _NEG, jnp.float32), segv)
    id_row, id_col, sc_row, sc_col, segv = jax.lax.fori_loop(
        0, _K, body, init)

    valid_row = lane_row < count
    valid_col = a_col < count
    rank_row = jnp.sum((valid_col & (id_col < id_row)).astype(jnp.int32),
                       axis=0, keepdims=True)
    place = (rank_row == a_col) & valid_row
    ordered_id_col = jnp.sum(jnp.where(place, id_row, 0), axis=1,
                             keepdims=True)
    ordered_sc_col = jnp.sum(jnp.where(place, sc_row, 0.0), axis=1,
                             keepdims=True)

    use_topk = count > _K
    fid_col = jnp.where(use_topk, id_col, ordered_id_col)
    fsc_col = jnp.where(use_topk, sc_col, ordered_sc_col)
    valid_out = a_col < jnp.minimum(count, _K)

    box_idx = fid_col % _N
    label = fid_col // _N
    i_row = jax.lax.broadcasted_iota(jnp.int32, (128, _N), 1)
    onehot = (box_idx == i_row).astype(jnp.float32)
    gat = jax.lax.dot_general(
        onehot, feat_ref[...], (((1,), (0,)), ((), ())),
        preferred_element_type=jnp.float32,
        precision=jax.lax.Precision.HIGHEST)
    featout_ref[...] = jnp.where(valid_out, gat, jnp.float32(-1.0))
    scoreout_ref[...] = jnp.where(valid_out, fsc_col, jnp.float32(-1.0))
    labelout_ref[...] = jnp.where(valid_out, label, jnp.int32(-1))


@jax.jit
def kernel(boxes, classification, rotation, translation):
    flat = classification.T.reshape(-1)
    flat = jnp.pad(flat, (0, _PAD), constant_values=-1.0)
    sc2d = flat.reshape(_R, _L)
    feat = jnp.concatenate([boxes, rotation, translation], axis=1)

    featout, scores, labels = pl.pallas_call(
        _filter_kernel,
        out_shape=[jax.ShapeDtypeStruct((128, 10), jnp.float32),
                   jax.ShapeDtypeStruct((128, 1), jnp.float32),
                   jax.ShapeDtypeStruct((128, 1), jnp.int32)],
        scratch_shapes=[pltpu.VMEM((_R, _L), jnp.float32)],
        compiler_params=pltpu.CompilerParams(
            vmem_limit_bytes=100 * 1024 * 1024),
    )(sc2d, feat)

    boxes_out = featout[:_K, :4]
    rotation_out = featout[:_K, 4:7]
    translation_out = featout[:_K, 7:10]
    scores_out = scores[:_K, 0]
    labels_out = labels[:_K, 0]
    return boxes_out, scores_out, labels_out, rotation_out, translation_out
